# baseline (speedup 1.0000x reference)
import functools

import jax
import jax.numpy as jnp
from jax import lax
from jax.experimental import pallas as pl
from jax.experimental.pallas import tpu as pltpu
from jax.experimental.pallas import tpu_sc as plsc

N = 10000
E = 320000
D = 128

_INFO = plsc.get_sparse_core_info()
NC = _INFO.num_cores
NS = _INFO.num_subcores
NW = NC * NS
NPAD = 10240
ROWS_PT = NPAD // NS
EPW = 10240
EPAD = EPW * NW

DCH = 128
DNCH = EPW // DCH

CH = 80
SNCH = EPW // CH
NGRP = SNCH // 2

def _worker_id():
    return lax.axis_index("s") * NC + lax.axis_index("c")


@functools.partial(
    pl.kernel,
    mesh=plsc.VectorSubcoreMesh(core_axis_name="c", subcore_axis_name="s"),
    out_type=jax.ShapeDtypeStruct((NC * NPAD,), jnp.float32),
    scratch_types=[
        pltpu.VMEM((DNCH, DCH), jnp.int32),
        pltpu.VMEM((ROWS_PT,), jnp.float32),
        pltpu.VMEM((DCH,), jnp.float32),
        pltpu.VMEM_SHARED((NPAD,), jnp.float32),
        pltpu.SemaphoreType.DMA,
    ],
)
def _sc_degree(dst_hbm, out_hbm, didx, vbuf, ones, acc, sem):
    cid = lax.axis_index("c")
    sid = lax.axis_index("s")
    wid = _worker_id()

    pltpu.sync_copy(dst_hbm.at[pl.ds(wid * DNCH, DNCH)], didx)

    def _fill(i, _):
        vbuf[pl.ds(i * 16, 16)] = jnp.zeros((16,), jnp.float32)
        return ()
    lax.fori_loop(0, ROWS_PT // 16, _fill, ())
    for j in range(DCH // 16):
        ones[pl.ds(j * 16, 16)] = jnp.ones((16,), jnp.float32)

    pltpu.sync_copy(vbuf, acc.at[pl.ds(sid * ROWS_PT, ROWS_PT)])
    plsc.subcore_barrier()

    LAG = 8

    def _body(g, _):
        pltpu.async_copy(ones, acc.at[didx.at[g]], sem, add=True)

        @pl.when(g >= LAG)
        def _():
            pltpu.make_async_copy(ones, acc.at[didx.at[0]], sem).wait()
        return ()
    lax.fori_loop(0, DNCH, _body, ())
    for _ in range(LAG):
        pltpu.make_async_copy(ones, acc.at[didx.at[0]], sem).wait()
    plsc.subcore_barrier()

    pltpu.sync_copy(acc.at[pl.ds(sid * ROWS_PT, ROWS_PT)], vbuf)
    pltpu.sync_copy(vbuf, out_hbm.at[pl.ds(cid * NPAD + sid * ROWS_PT, ROWS_PT)])


@functools.partial(
    pl.kernel,
    mesh=plsc.VectorSubcoreMesh(core_axis_name="c", subcore_axis_name="s"),
    out_type=jax.ShapeDtypeStruct((NC * NPAD, D), jnp.float32),
    scratch_types=[
        pltpu.VMEM((4, CH), jnp.int32),
        pltpu.VMEM((4, CH), jnp.int32),
        pltpu.VMEM((4 * CH, D), jnp.float32),
        pltpu.VMEM_SHARED((NPAD, D), jnp.float32),
    ] + [pltpu.SemaphoreType.DMA] * 12,
)
def _sc_scatter(p_hbm, src_hbm, dst_hbm, out_hbm, sidx, didx, rows, acc,
                *sems):
    gsems, ssems = sems[0:4], sems[4:8]
    isrc, idst = sems[8:10], sems[10:12]
    cid = lax.axis_index("c")
    sid = lax.axis_index("s")
    wid = _worker_id()

    def _fill(i, _):
        for j in range(D // 16):
            rows[i, pl.ds(j * 16, 16)] = jnp.zeros((16,), jnp.float32)
        return ()
    lax.fori_loop(0, 4 * CH, _fill, ())
    for t in range(ROWS_PT // (4 * CH)):
        pltpu.sync_copy(rows.at[pl.ds(0, 4 * CH)],
                        acc.at[pl.ds(sid * ROWS_PT + t * 4 * CH, 4 * CH)])
    plsc.subcore_barrier()

    gbase = wid * SNCH

    def _fire_idx_src(g, slot):
        pltpu.async_copy(src_hbm.at[pl.ds(gbase + g * 2, 2)],
                         sidx.at[pl.ds(slot * 2, 2)], isrc[slot])

    def _fire_idx_dst(g, slot):
        pltpu.async_copy(dst_hbm.at[pl.ds(gbase + g * 2, 2)],
                         didx.at[pl.ds(slot * 2, 2)], idst[slot])

    def _wait_idx(sem_arr, buf, slot):
        pltpu.make_async_copy(src_hbm.at[pl.ds(0, 2)],
                              buf.at[pl.ds(slot * 2, 2)], sem_arr[slot]).wait()

    def _fire_gather(k):
        pltpu.async_copy(p_hbm.at[sidx.at[k]],
                         rows.at[pl.ds(k * CH, CH)], gsems[k])

    def _wait_gather(k):
        pltpu.make_async_copy(p_hbm.at[sidx.at[k]],
                              rows.at[pl.ds(k * CH, CH)], gsems[k]).wait()

    def _fire_scatter(k):
        pltpu.async_copy(rows.at[pl.ds(k * CH, CH)],
                         acc.at[didx.at[k]], ssems[k], add=True)

    def _wait_scatter(k):
        pltpu.make_async_copy(rows.at[pl.ds(k * CH, CH)],
                              acc.at[didx.at[k]], ssems[k]).wait()

    _fire_idx_src(0, 0)
    _fire_idx_dst(0, 0)
    _wait_idx(isrc, sidx, 0)
    _fire_gather(0)
    _fire_gather(1)

    def _iter(it, _):
        grp = 2 * it
        _fire_idx_src(grp + 1, 1)
        _wait_gather(0)
        _wait_gather(1)
        _wait_idx(idst, didx, 0)
        _fire_scatter(0)
        _fire_scatter(1)

        @pl.when(it > 0)
        def _():
            _wait_scatter(2)
            _wait_scatter(3)

        _fire_idx_dst(grp + 1, 1)
        _wait_idx(isrc, sidx, 1)
        _fire_gather(2)
        _fire_gather(3)

        @pl.when(it < NGRP // 2 - 1)
        def _():
            _fire_idx_src(grp + 2, 0)
        _wait_gather(2)
        _wait_gather(3)
        _wait_idx(idst, didx, 1)
        _fire_scatter(2)
        _fire_scatter(3)
        _wait_scatter(0)
        _wait_scatter(1)

        @pl.when(it < NGRP // 2 - 1)
        def _():
            _fire_idx_dst(grp + 2, 0)
            _wait_idx(isrc, sidx, 0)
            _fire_gather(0)
            _fire_gather(1)
        return ()
    lax.fori_loop(0, NGRP // 2, _iter, ())
    _wait_scatter(2)
    _wait_scatter(3)
    plsc.subcore_barrier()

    obase = cid * NPAD + sid * ROWS_PT
    for t in range(ROWS_PT // (4 * CH)):
        pltpu.sync_copy(acc.at[pl.ds(sid * ROWS_PT + t * 4 * CH, 4 * CH)],
                        rows.at[pl.ds(0, 4 * CH)])
        pltpu.sync_copy(rows.at[pl.ds(0, 4 * CH)],
                        out_hbm.at[pl.ds(obase + t * 4 * CH, 4 * CH)])


RB = 2000


def _ln_relu_res(acc2, p, h_prev, dis, bc, g, b):
    t = (acc2 + p) * dis + bc
    mu = jnp.mean(t, axis=-1, keepdims=True)
    var = jnp.mean((t - mu) * (t - mu), axis=-1, keepdims=True)
    ln = (t - mu) * lax.rsqrt(var + 1e-5) * g + b
    return jnp.maximum(ln, 0.0) + h_prev


def _k0_body(x_ref, win_ref, bin_ref, wc0_ref, deg_ref, h_ref, p_ref,
             dis_ref):
    h = jnp.dot(x_ref[...], win_ref[...], preferred_element_type=jnp.float32)
    h = jnp.maximum(h + bin_ref[...], 0.0)
    h_ref[...] = h
    dis = lax.rsqrt(deg_ref[0] + deg_ref[1] + 1.0)
    dis_ref[...] = dis
    p = jnp.dot(h, wc0_ref[...], preferred_element_type=jnp.float32)
    p_ref[...] = p * dis


def _klayer_body(acc_ref, p_ref, h_ref, dis_ref, bc_ref, g_ref, b_ref, w_ref,
                 hout_ref, pout_ref):
    dis = dis_ref[...]
    h = _ln_relu_res(acc_ref[0] + acc_ref[1], p_ref[...], h_ref[...], dis,
                     bc_ref[...], g_ref[...], b_ref[...])
    hout_ref[...] = h
    p = jnp.dot(h, w_ref[...], preferred_element_type=jnp.float32)
    pout_ref[...] = p * dis


def _kfinal_body(acc_ref, p_ref, h_ref, dis_ref, bc_ref, g_ref, b_ref,
                 wout_ref, bout_ref, out_ref):
    h = _ln_relu_res(acc_ref[0] + acc_ref[1], p_ref[...], h_ref[...],
                     dis_ref[...], bc_ref[...], g_ref[...], b_ref[...])
    out = jnp.dot(h, wout_ref[...], preferred_element_type=jnp.float32)
    out_ref[...] = out + bout_ref[...]


_row_spec = pl.BlockSpec((RB, D), lambda i: (i, 0))
_mat_spec = pl.BlockSpec((D, D), lambda i: (0, 0))
_vec_spec = pl.BlockSpec((1, D), lambda i: (0, 0))
_dis_spec = pl.BlockSpec((RB, 1), lambda i: (i, 0))
_acc_spec = pl.BlockSpec((2, RB, D), lambda i: (0, i, 0))
_GRID = (N // RB,)
_out2 = [jax.ShapeDtypeStruct((N, D), jnp.float32)] * 2


_deg_spec = pl.BlockSpec((2, RB, 1), lambda i: (0, i, 0))


def _tc_k0(x, W_in, b_in2, Wc0, deg2):
    return pl.pallas_call(
        _k0_body, grid=_GRID,
        in_specs=[_row_spec, _mat_spec, _vec_spec, _mat_spec, _deg_spec],
        out_specs=[_row_spec, _row_spec, _dis_spec],
        out_shape=_out2 + [jax.ShapeDtypeStruct((N, 1), jnp.float32)],
    )(x, W_in, b_in2, Wc0, deg2)


def _tc_layer(acc3, p, h, dis_col, bc2, g2, b2, Wn):
    return pl.pallas_call(
        _klayer_body, grid=_GRID,
        in_specs=[_acc_spec, _row_spec, _row_spec, _dis_spec,
                  _vec_spec, _vec_spec, _vec_spec, _mat_spec],
        out_specs=[_row_spec, _row_spec], out_shape=_out2,
    )(acc3, p, h, dis_col, bc2, g2, b2, Wn)


def _tc_final(acc3, p, h, dis_col, bc2, g2, b2, W_out, b_out2):
    return pl.pallas_call(
        _kfinal_body, grid=_GRID,
        in_specs=[_acc_spec, _row_spec, _row_spec, _dis_spec,
                  _vec_spec, _vec_spec, _vec_spec, _mat_spec, _vec_spec],
        out_specs=_row_spec, out_shape=jax.ShapeDtypeStruct((N, D), jnp.float32),
    )(acc3, p, h, dis_col, bc2, g2, b2, W_out, b_out2)


def kernel(x, edge_index, W_in, b_in, Wc, bc, gamma, beta, W_out, b_out):
    src = edge_index[0].astype(jnp.int32)
    dst = edge_index[1].astype(jnp.int32)
    npad_e = EPAD - E
    pad_src = jnp.arange(npad_e, dtype=jnp.int32) % N
    src_f = jnp.concatenate([src, pad_src])
    pad_dst = N + (jnp.arange(npad_e, dtype=jnp.int32) % (NPAD - N))
    dst_f = jnp.concatenate([dst, pad_dst])
    src_p = src_f.reshape(EPAD // CH, CH)
    dst_p = dst_f.reshape(EPAD // CH, CH)

    deg2 = _sc_degree(dst_f.reshape(EPAD // DCH, DCH)).reshape(2, NPAD, 1)

    h, p, dis_col = _tc_k0(x, W_in, b_in[None, :], Wc[0], deg2)
    for i in range(Wc.shape[0]):
        acc = _sc_scatter(p, src_p, dst_p).reshape(NC, NPAD, D)
        args = (acc, p, h, dis_col, bc[i][None, :], gamma[i][None, :],
                beta[i][None, :])
        if i + 1 < Wc.shape[0]:
            h, p = _tc_layer(*args, Wc[i + 1])
        else:
            return _tc_final(*args, W_out, b_out[None, :])


# speedup vs baseline: 1.0164x; 1.0164x over previous
import functools

import jax
import jax.numpy as jnp
from jax import lax
from jax.experimental import pallas as pl
from jax.experimental.pallas import tpu as pltpu
from jax.experimental.pallas import tpu_sc as plsc

N = 10000
E = 320000
D = 128

_INFO = plsc.get_sparse_core_info()
NC = _INFO.num_cores
NS = _INFO.num_subcores
NW = NC * NS

NPAD = 10240
EPAD = 327680
ROWS_PT = NPAD // NS
EPW = EPAD // NW

DEPW = E // NW
DCH = 125
DNCH = DEPW // DCH

CH = 80
SNCH = EPW // CH
NGRP = SNCH // 2


def _worker_id():
    return lax.axis_index("s") * NC + lax.axis_index("c")


@functools.partial(
    pl.kernel,
    mesh=plsc.VectorSubcoreMesh(core_axis_name="c", subcore_axis_name="s"),
    out_type=jax.ShapeDtypeStruct((NC * NPAD,), jnp.float32),
    scratch_types=[
        pltpu.VMEM((DNCH, DCH), jnp.int32),
        pltpu.VMEM((ROWS_PT,), jnp.float32),
        pltpu.VMEM((128,), jnp.float32),
        pltpu.VMEM_SHARED((NPAD,), jnp.float32),
        pltpu.SemaphoreType.DMA,
    ],
)
def _sc_degree(dst_hbm, out_hbm, didx, vbuf, ones, acc, sem):
    cid = lax.axis_index("c")
    sid = lax.axis_index("s")
    wid = _worker_id()

    pltpu.sync_copy(dst_hbm.at[pl.ds(wid * DNCH, DNCH)], didx)

    def _fill(i, _):
        vbuf[pl.ds(i * 16, 16)] = jnp.zeros((16,), jnp.float32)
        return ()
    lax.fori_loop(0, ROWS_PT // 16, _fill, ())
    for j in range(128 // 16):
        ones[pl.ds(j * 16, 16)] = jnp.ones((16,), jnp.float32)

    pltpu.sync_copy(vbuf.at[pl.ds(0, ROWS_PT)],
                    acc.at[pl.ds(sid * ROWS_PT, ROWS_PT)])
    plsc.subcore_barrier()

    LAG = 8

    def _body(g, _):
        pltpu.async_copy(ones.at[pl.ds(0, DCH)], acc.at[didx.at[g]], sem,
                         add=True)

        @pl.when(g >= LAG)
        def _():
            pltpu.make_async_copy(ones.at[pl.ds(0, DCH)], acc.at[didx.at[0]],
                                  sem).wait()
        return ()
    lax.fori_loop(0, DNCH, _body, ())
    for _ in range(LAG):
        pltpu.make_async_copy(ones.at[pl.ds(0, DCH)], acc.at[didx.at[0]],
                              sem).wait()
    plsc.subcore_barrier()

    pltpu.sync_copy(acc.at[pl.ds(sid * ROWS_PT, ROWS_PT)],
                    vbuf.at[pl.ds(0, ROWS_PT)])
    pltpu.sync_copy(vbuf.at[pl.ds(0, ROWS_PT)],
                    out_hbm.at[pl.ds(cid * NPAD + sid * ROWS_PT, ROWS_PT)])


@functools.partial(
    pl.kernel,
    mesh=plsc.VectorSubcoreMesh(core_axis_name="c", subcore_axis_name="s"),
    out_type=jax.ShapeDtypeStruct((NC * NPAD, D), jnp.float32),
    scratch_types=[
        pltpu.VMEM((4, CH), jnp.int32),
        pltpu.VMEM((4, CH), jnp.int32),
        pltpu.VMEM((4 * CH, D), jnp.float32),
        pltpu.VMEM_SHARED((NPAD, D), jnp.float32),
    ] + [pltpu.SemaphoreType.DMA] * 12,
)
def _sc_scatter(p_hbm, src_hbm, dst_hbm, out_hbm, sidx, didx, rows, acc,
                *sems):
    gsems, ssems = sems[0:4], sems[4:8]
    isrc, idst = sems[8:10], sems[10:12]
    cid = lax.axis_index("c")
    sid = lax.axis_index("s")
    wid = _worker_id()

    def _fill(i, _):
        for j in range(D // 16):
            rows[i, pl.ds(j * 16, 16)] = jnp.zeros((16,), jnp.float32)
        return ()
    lax.fori_loop(0, 4 * CH, _fill, ())
    for t in range(5):
        pltpu.sync_copy(rows.at[pl.ds(0, ROWS_PT // 5)],
                        acc.at[pl.ds(sid * ROWS_PT + t * (ROWS_PT // 5),
                                     ROWS_PT // 5)])
    plsc.subcore_barrier()

    gbase = wid * SNCH

    def _fire_idx_src(g, slot):
        pltpu.async_copy(src_hbm.at[pl.ds(gbase + g * 2, 2)],
                         sidx.at[pl.ds(slot * 2, 2)], isrc[slot])

    def _fire_idx_dst(g, slot):
        pltpu.async_copy(dst_hbm.at[pl.ds(gbase + g * 2, 2)],
                         didx.at[pl.ds(slot * 2, 2)], idst[slot])

    def _wait_idx(sem_arr, buf, slot):
        pltpu.make_async_copy(src_hbm.at[pl.ds(0, 2)],
                              buf.at[pl.ds(slot * 2, 2)], sem_arr[slot]).wait()

    def _fire_gather(k):
        pltpu.async_copy(p_hbm.at[sidx.at[k]],
                         rows.at[pl.ds(k * CH, CH)], gsems[k])

    def _wait_gather(k):
        pltpu.make_async_copy(p_hbm.at[sidx.at[k]],
                              rows.at[pl.ds(k * CH, CH)], gsems[k]).wait()

    def _fire_scatter(k):
        pltpu.async_copy(rows.at[pl.ds(k * CH, CH)],
                         acc.at[didx.at[k]], ssems[k], add=True)

    def _wait_scatter(k):
        pltpu.make_async_copy(rows.at[pl.ds(k * CH, CH)],
                              acc.at[didx.at[k]], ssems[k]).wait()

    _fire_idx_src(0, 0)
    _fire_idx_dst(0, 0)
    _wait_idx(isrc, sidx, 0)
    _fire_gather(0)
    _fire_gather(1)

    def _iter(it, _):
        grp = 2 * it
        _fire_idx_src(grp + 1, 1)
        _wait_gather(0)
        _wait_gather(1)
        _wait_idx(idst, didx, 0)
        _fire_scatter(0)
        _fire_scatter(1)

        @pl.when(it > 0)
        def _():
            _wait_scatter(2)
            _wait_scatter(3)

        _fire_idx_dst(grp + 1, 1)
        _wait_idx(isrc, sidx, 1)
        _fire_gather(2)
        _fire_gather(3)

        @pl.when(it < NGRP // 2 - 1)
        def _():
            _fire_idx_src(grp + 2, 0)
        _wait_gather(2)
        _wait_gather(3)
        _wait_idx(idst, didx, 1)
        _fire_scatter(2)
        _fire_scatter(3)
        _wait_scatter(0)
        _wait_scatter(1)

        @pl.when(it < NGRP // 2 - 1)
        def _():
            _fire_idx_dst(grp + 2, 0)
            _wait_idx(isrc, sidx, 0)
            _fire_gather(0)
            _fire_gather(1)
        return ()
    lax.fori_loop(0, NGRP // 2, _iter, ())
    _wait_scatter(2)
    _wait_scatter(3)
    plsc.subcore_barrier()

    obase = cid * NPAD + sid * ROWS_PT
    for t in range(ROWS_PT // (4 * CH)):
        pltpu.sync_copy(acc.at[pl.ds(sid * ROWS_PT + t * 4 * CH, 4 * CH)],
                        rows.at[pl.ds(0, 4 * CH)])
        pltpu.sync_copy(rows.at[pl.ds(0, 4 * CH)],
                        out_hbm.at[pl.ds(obase + t * 4 * CH, 4 * CH)])


RB = 2000


def _ln_relu_res(acc2, p, h_prev, dis, bc, g, b):
    t = (acc2 + p) * dis + bc
    mu = jnp.mean(t, axis=-1, keepdims=True)
    var = jnp.mean((t - mu) * (t - mu), axis=-1, keepdims=True)
    ln = (t - mu) * lax.rsqrt(var + 1e-5) * g + b
    return jnp.maximum(ln, 0.0) + h_prev


def _k0_body(x_ref, win_ref, bin_ref, wc0_ref, dis_ref, h_ref, p_ref):
    h = jnp.dot(x_ref[...], win_ref[...], preferred_element_type=jnp.float32)
    h = jnp.maximum(h + bin_ref[...], 0.0)
    h_ref[...] = h
    p = jnp.dot(h, wc0_ref[...], preferred_element_type=jnp.float32)
    p_ref[...] = p * dis_ref[...]


def _klayer_body(acc_ref, p_ref, h_ref, dis_ref, bc_ref, g_ref, b_ref, w_ref,
                 hout_ref, pout_ref):
    dis = dis_ref[...]
    h = _ln_relu_res(acc_ref[0] + acc_ref[1], p_ref[...], h_ref[...], dis,
                     bc_ref[...], g_ref[...], b_ref[...])
    hout_ref[...] = h
    p = jnp.dot(h, w_ref[...], preferred_element_type=jnp.float32)
    pout_ref[...] = p * dis


def _kfinal_body(acc_ref, p_ref, h_ref, dis_ref, bc_ref, g_ref, b_ref,
                 wout_ref, bout_ref, out_ref):
    h = _ln_relu_res(acc_ref[0] + acc_ref[1], p_ref[...], h_ref[...],
                     dis_ref[...], bc_ref[...], g_ref[...], b_ref[...])
    out = jnp.dot(h, wout_ref[...], preferred_element_type=jnp.float32)
    out_ref[...] = out + bout_ref[...]


_row_spec = pl.BlockSpec((RB, D), lambda i: (i, 0))
_mat_spec = pl.BlockSpec((D, D), lambda i: (0, 0))
_vec_spec = pl.BlockSpec((1, D), lambda i: (0, 0))
_dis_spec = pl.BlockSpec((RB, 1), lambda i: (i, 0))
_acc_spec = pl.BlockSpec((2, RB, D), lambda i: (0, i, 0))
_GRID = (N // RB,)
_out2 = [jax.ShapeDtypeStruct((N, D), jnp.float32)] * 2


def _tc_k0(x, W_in, b_in2, Wc0, dis_col):
    return pl.pallas_call(
        _k0_body, grid=_GRID,
        in_specs=[_row_spec, _mat_spec, _vec_spec, _mat_spec, _dis_spec],
        out_specs=[_row_spec, _row_spec], out_shape=_out2,
    )(x, W_in, b_in2, Wc0, dis_col)


def _tc_layer(acc3, p, h, dis_col, bc2, g2, b2, Wn):
    return pl.pallas_call(
        _klayer_body, grid=_GRID,
        in_specs=[_acc_spec, _row_spec, _row_spec, _dis_spec,
                  _vec_spec, _vec_spec, _vec_spec, _mat_spec],
        out_specs=[_row_spec, _row_spec], out_shape=_out2,
    )(acc3, p, h, dis_col, bc2, g2, b2, Wn)


def _tc_final(acc3, p, h, dis_col, bc2, g2, b2, W_out, b_out2):
    return pl.pallas_call(
        _kfinal_body, grid=_GRID,
        in_specs=[_acc_spec, _row_spec, _row_spec, _dis_spec,
                  _vec_spec, _vec_spec, _vec_spec, _mat_spec, _vec_spec],
        out_specs=_row_spec, out_shape=jax.ShapeDtypeStruct((N, D), jnp.float32),
    )(acc3, p, h, dis_col, bc2, g2, b2, W_out, b_out2)


def kernel(x, edge_index, W_in, b_in, Wc, bc, gamma, beta, W_out, b_out):
    src = edge_index[0].astype(jnp.int32)
    dst = edge_index[1].astype(jnp.int32)
    npad_e = EPAD - E
    pad_src = jnp.arange(npad_e, dtype=jnp.int32) % N
    pad_dst = N + jnp.arange(npad_e, dtype=jnp.int32) % (NPAD - N)
    src_p = jnp.concatenate([src, pad_src]).reshape(EPAD // CH, CH)
    dst_p = jnp.concatenate([dst, pad_dst]).reshape(EPAD // CH, CH)

    deg2 = _sc_degree(dst.reshape(E // DCH, DCH)).reshape(NC, NPAD)
    dis_col = lax.rsqrt(deg2[0, :N] + deg2[1, :N] + 1.0).reshape(N, 1)

    h, p = _tc_k0(x, W_in, b_in[None, :], Wc[0], dis_col)
    for i in range(Wc.shape[0]):
        acc = _sc_scatter(p, src_p, dst_p).reshape(NC, NPAD, D)
        args = (acc, p, h, dis_col, bc[i][None, :], gamma[i][None, :],
                beta[i][None, :])
        if i + 1 < Wc.shape[0]:
            h, p = _tc_layer(*args, Wc[i + 1])
        else:
            return _tc_final(*args, W_out, b_out[None, :])
